# baseline (speedup 1.0000x reference)
import functools

import jax
import jax.numpy as jnp
import numpy as np
from jax.experimental import pallas as pl
from jax.experimental.pallas import tpu as pltpu

_H = 16
_NEG = -1e30
_QSCALE = float(np.log2(np.e) / np.sqrt(128.0))


def _qkv_kernel(x_ref, wq_ref, wk_ref, wv_ref, q_ref, k_ref, v_ref):
    x = x_ref[...].astype(jnp.bfloat16)
    q_ref[...] = (jnp.dot(x, wq_ref[...].astype(jnp.bfloat16),
                          preferred_element_type=jnp.float32)
                  * _QSCALE).astype(jnp.bfloat16)
    k_ref[...] = jnp.dot(x, wk_ref[...].astype(jnp.bfloat16),
                         preferred_element_type=jnp.float32).astype(jnp.bfloat16)
    v_ref[...] = jnp.dot(x, wv_ref[...].astype(jnp.bfloat16),
                         preferred_element_type=jnp.float32).astype(jnp.bfloat16)


def _qkv_matmul(x, wq, wk, wv, bm, bn):
    m, d = x.shape
    grid = (m // bm, d // bn)
    wspec = pl.BlockSpec((d, bn), lambda i, j: (0, j))
    ospec = pl.BlockSpec((bm, bn), lambda i, j: (i, j))
    osds = jax.ShapeDtypeStruct((m, d), jnp.bfloat16)
    return pl.pallas_call(
        _qkv_kernel,
        grid=grid,
        in_specs=[pl.BlockSpec((bm, d), lambda i, j: (i, 0)),
                  wspec, wspec, wspec],
        out_specs=[ospec, ospec, ospec],
        out_shape=[osds, osds, osds],
        compiler_params=pltpu.CompilerParams(
            dimension_semantics=("parallel", "parallel"),
            vmem_limit_bytes=57 * 1024 * 1024,
        ),
    )(x, wq, wk, wv)


def _attn_kernel(q_ref, k_ref, v_ref, o_ref, *, bq, bk):
    S = q_ref.shape[1]
    nq = S // bq
    for qi in range(nq):
        q = q_ref[0, qi * bq:(qi + 1) * bq, :]
        o_acc = m = l = None
        for j in range((qi + 1) * bq // bk):
            k_blk = k_ref[0, j * bk:(j + 1) * bk, :]
            s = jax.lax.dot_general(
                q, k_blk, (((1,), (1,)), ((), ())),
                preferred_element_type=jnp.float32,
            )
            if (j + 1) * bk > qi * bq + 1:
                rows = jax.lax.broadcasted_iota(jnp.int32, (bq, bk), 0) + qi * bq
                cols = jax.lax.broadcasted_iota(jnp.int32, (bq, bk), 1) + j * bk
                s = jnp.where(cols <= rows, s, _NEG)
            v_blk = v_ref[0, j * bk:(j + 1) * bk, :]
            if j == 0:
                m = jnp.max(s, axis=-1, keepdims=True)
                p = jnp.exp2(s - m)
                l = jnp.sum(p, axis=-1, keepdims=True)
                o_acc = jax.lax.dot_general(
                    p.astype(jnp.bfloat16), v_blk, (((1,), (0,)), ((), ())),
                    preferred_element_type=jnp.float32,
                )
            else:
                m_new = jnp.maximum(m, jnp.max(s, axis=-1, keepdims=True))
                alpha = jnp.exp2(m - m_new)
                p = jnp.exp2(s - m_new)
                l = l * alpha + jnp.sum(p, axis=-1, keepdims=True)
                pv = jax.lax.dot_general(
                    p.astype(jnp.bfloat16), v_blk, (((1,), (0,)), ((), ())),
                    preferred_element_type=jnp.float32,
                )
                o_acc = o_acc * alpha + pv
                m = m_new
        o_ref[0, qi * bq:(qi + 1) * bq, :] = (o_acc / l).astype(jnp.bfloat16)


def _attention(q, k, v, bq, bk):
    B, S, D = q.shape
    dk = D // _H
    kern = functools.partial(_attn_kernel, bq=bq, bk=bk)
    hspec = pl.BlockSpec((1, S, dk), lambda b, h: (b, 0, h))
    return pl.pallas_call(
        kern,
        grid=(B, _H),
        in_specs=[hspec, hspec, hspec],
        out_specs=hspec,
        out_shape=jax.ShapeDtypeStruct((B, S, D), jnp.bfloat16),
        compiler_params=pltpu.CompilerParams(
            dimension_semantics=("parallel", "arbitrary"),
            vmem_limit_bytes=50 * 1024 * 1024,
        ),
    )(q, k, v)


def _out_kernel(o_ref, w_ref, out_ref):
    out_ref[...] = jnp.dot(o_ref[...], w_ref[...].astype(jnp.bfloat16),
                           preferred_element_type=jnp.float32)


def _out_matmul(o, w, bm, bn):
    m, d = o.shape
    return pl.pallas_call(
        _out_kernel,
        grid=(m // bm, d // bn),
        in_specs=[pl.BlockSpec((bm, d), lambda i, j: (i, 0)),
                  pl.BlockSpec((d, bn), lambda i, j: (0, j))],
        out_specs=pl.BlockSpec((bm, bn), lambda i, j: (i, j)),
        out_shape=jax.ShapeDtypeStruct((m, d), jnp.float32),
        compiler_params=pltpu.CompilerParams(
            dimension_semantics=("parallel", "parallel"),
            vmem_limit_bytes=50 * 1024 * 1024,
        ),
    )(o, w)


def kernel(x, Wq, Wk, Wv, Wo):
    B, S, D = x.shape
    x2d = x.reshape(B * S, D).astype(jnp.bfloat16)
    q, k, v = _qkv_matmul(x2d, Wq, Wk, Wv, bm=2048, bn=256)
    q = q.reshape(B, S, D)
    k = k.reshape(B, S, D)
    v = v.reshape(B, S, D)
    o = _attention(q, k, v, bq=512, bk=256)
    out = _out_matmul(o.reshape(B * S, D), Wo, bm=2048, bn=512)
    return out.reshape(B, S, D)
